# baseline (speedup 1.0000x reference)
import functools

import jax
import jax.numpy as jnp
from jax import lax
from jax.experimental import pallas as pl
from jax.experimental.pallas import tpu as pltpu
from jax.experimental.pallas import tpu_sc as plsc

_DIM = 2048
_E = 16
_TOKENS = 16384
_FLAT = _TOKENS * 2
_NH = 32
_BT = 2048
_GRID = _TOKENS // _BT
_HPB = _BT // 512

_SC_W = 16
_CHUNK = _FLAT // _SC_W
_COLS = 128
_ROWS = _CHUNK // _COLS


def _router_block(x_ref, wt_ref, bias_ref, sco_ref, sel_ref, hist_ref):
    x = x_ref[...]
    logits = jnp.dot(x, wt_ref[...], preferred_element_type=jnp.float32)
    logits = logits + bias_ref[...]
    eio = lax.broadcasted_iota(jnp.int32, (_BT, _E), 1)
    m1 = jnp.max(logits, axis=1, keepdims=True)
    i1 = jnp.min(jnp.where(logits >= m1, eio, _E), axis=1, keepdims=True)
    masked = jnp.where(eio == i1, -jnp.inf, logits)
    m2 = jnp.max(masked, axis=1, keepdims=True)
    i2 = jnp.min(jnp.where(masked >= m2, eio, _E), axis=1, keepdims=True)
    t = jnp.exp(m2 - m1)
    s = 1.0 / (1.0 + t)
    sco_ref[...] = jnp.concatenate([s, t * s], axis=1)
    sel_ref[...] = jnp.concatenate([i1, i2], axis=1)
    oh = (i1 == eio).astype(jnp.int32) + (i2 == eio).astype(jnp.int32)
    hs = [jnp.sum(oh[c * 512 : (c + 1) * 512], axis=0, keepdims=True)
          for c in range(_HPB)]
    hist_ref[...] = jnp.concatenate(hs, axis=0)[None]


def _router_tc(x, wt, bias2d):
    return pl.pallas_call(
        _router_block,
        grid=(_GRID,),
        in_specs=[
            pl.BlockSpec((_BT, _DIM), lambda i: (i, 0)),
            pl.BlockSpec((_DIM, _E), lambda i: (0, 0)),
            pl.BlockSpec((1, _E), lambda i: (0, 0)),
        ],
        out_specs=[
            pl.BlockSpec((_BT, 2), lambda i: (i, 0)),
            pl.BlockSpec((_BT, 2), lambda i: (i, 0)),
            pl.BlockSpec((1, _HPB, _E), lambda i: (i, 0, 0)),
        ],
        out_shape=[
            jax.ShapeDtypeStruct((_TOKENS, 2), jnp.float32),
            jax.ShapeDtypeStruct((_TOKENS, 2), jnp.int32),
            jax.ShapeDtypeStruct((_GRID, _HPB, _E), jnp.int32),
        ],
    )(x, wt, bias2d)


def _dispatch_body(sel_hbm, sco_hbm, hist_hbm, sco_out, tid_out, cnt_out,
                   sel_v, sco_v, pos_v, tid_v, hist_v, se_b, sl_b, r_b, h_b,
                   rs_b, cnt_v, sco_sh, tid_sh, sem, sem_h, sem_s):
    cid = lax.axis_index("c")
    wid = lax.axis_index("s")
    cp_h = pltpu.make_async_copy(hist_hbm, hist_v, sem_h)
    cp_s = pltpu.make_async_copy(sel_hbm.at[wid], sel_v, sem_s)
    cp_v = pltpu.make_async_copy(sco_hbm.at[wid], sco_v, sem)
    cp_h.start()
    cp_s.start()

    @pl.when(cid == 0)
    def _():
        cp_v.start()

    cp_h.wait()
    tot = jnp.zeros((_E,), jnp.int32)
    pre = jnp.zeros((_E,), jnp.int32)
    for c in range(_NH):
        h = hist_v[c, :]
        tot = tot + h
        pre = pre + jnp.where(c < wid * 2, h, 0)
    excl = lax.cumsum(tot, axis=0) - tot

    @pl.when((wid == 0) & (cid == 0))
    def _():
        cnt_v[...] = tot
        pltpu.sync_copy(cnt_v, cnt_out)
    cp_s.wait()

    lane = lax.iota(jnp.int32, 16)
    lane_m1 = jnp.maximum(lane - 1, 0)
    lane_p1 = jnp.minimum(lane + 1, 15)
    flat_base = wid * _CHUNK
    nvr = _CHUNK // 16

    @plsc.parallel_loop(0, nvr, unroll=8)
    def _(k):
        off = pl.ds(pl.multiple_of(k * 16, 16), 16)
        e = sel_v[off]
        sk, sl = plsc.sort_key_val(e * 16 + lane, lane)
        se = sk >> 4
        se_b[off] = se
        k16 = k * 16
        prev = plsc.load_gather(se_b, [k16 + lane_m1])
        nxt = plsc.load_gather(se_b, [k16 + lane_p1])
        is_start = (lane == 0) | (prev != se)
        is_end = (lane == 15) | (nxt != se)
        runstart = plsc.cummax(jnp.where(is_start, lane, 0))
        r = lane - runstart
        sl_b[off] = sl
        r_b[off] = r
        h_b[off] = jnp.zeros((16,), jnp.int32)
        plsc.store_scatter(h_b, [k16 + se], r + 1, mask=is_end)
        tid_v[off] = (flat_base + k16 + lane) >> 1

    def pre_body(k, acc):
        off = pl.ds(pl.multiple_of(k * 16, 16), 16)
        rs_b[off] = acc
        return acc + h_b[off]

    lax.fori_loop(0, nvr, pre_body, excl + pre)

    @plsc.parallel_loop(0, nvr, unroll=8)
    def _(k):
        off = pl.ds(pl.multiple_of(k * 16, 16), 16)
        k16 = k * 16
        se = se_b[off]
        base = plsc.load_gather(rs_b, [k16 + se])
        pos = base + r_b[off]
        row = jnp.zeros((16,), jnp.int32) + (k >> 3)
        col = (k & 7) * 16 + sl_b[off]
        plsc.store_scatter(pos_v, [row, col], pos)

    sl_out = pl.ds(wid * _CHUNK, _CHUNK)

    @pl.when(cid == 0)
    def _():
        cp_v.wait()
        copies = [pltpu.make_async_copy(
            sco_v.at[pl.ds(j * _COLS, _COLS)], sco_sh.at[pos_v.at[j]], sem)
            for j in range(_ROWS)]
        for cp in copies:
            cp.start()
        for cp in copies:
            cp.wait()
        plsc.subcore_barrier()
        pltpu.sync_copy(sco_sh.at[sl_out], sco_out.at[sl_out])

    @pl.when(cid != 0)
    def _():
        copies = [pltpu.make_async_copy(
            tid_v.at[pl.ds(j * _COLS, _COLS)], tid_sh.at[pos_v.at[j]], sem)
            for j in range(_ROWS)]
        for cp in copies:
            cp.start()
        for cp in copies:
            cp.wait()
        plsc.subcore_barrier()
        pltpu.sync_copy(tid_sh.at[sl_out], tid_out.at[sl_out])


def _dispatch_sc(sel2, sco2, hist):
    mesh = plsc.VectorSubcoreMesh(
        core_axis_name="c", subcore_axis_name="s", num_cores=2, num_subcores=16)
    run = pl.kernel(
        _dispatch_body,
        out_type=[
            jax.ShapeDtypeStruct((_FLAT,), jnp.float32),
            jax.ShapeDtypeStruct((_FLAT,), jnp.int32),
            jax.ShapeDtypeStruct((_E,), jnp.int32),
        ],
        mesh=mesh,
        compiler_params=pltpu.CompilerParams(needs_layout_passes=False),
        scratch_types=[
            pltpu.VMEM((_CHUNK,), jnp.int32),
            pltpu.VMEM((_CHUNK,), jnp.float32),
            pltpu.VMEM((_ROWS, _COLS), jnp.int32),
            pltpu.VMEM((_CHUNK,), jnp.int32),
            pltpu.VMEM((_NH, _E), jnp.int32),
            pltpu.VMEM((_CHUNK,), jnp.int32),
            pltpu.VMEM((_CHUNK,), jnp.int32),
            pltpu.VMEM((_CHUNK,), jnp.int32),
            pltpu.VMEM((_CHUNK,), jnp.int32),
            pltpu.VMEM((_CHUNK,), jnp.int32),
            pltpu.VMEM((_E,), jnp.int32),
            pltpu.VMEM_SHARED((_FLAT,), jnp.float32),
            pltpu.VMEM_SHARED((_FLAT,), jnp.int32),
            pltpu.SemaphoreType.DMA,
            pltpu.SemaphoreType.DMA,
            pltpu.SemaphoreType.DMA,
        ],
    )
    return run(sel2, sco2, hist)


def kernel(x, expert_bias, W, b):
    wt = W.T
    bias2d = (b + expert_bias).reshape(1, _E)
    sco, sel, hist = _router_tc(x, wt, bias2d)
    sel2 = sel.reshape(_SC_W, _CHUNK)
    sco2 = sco.reshape(_SC_W, _CHUNK)
    sco_sorted, tid_sorted, counts = _dispatch_sc(
        sel2, sco2, hist.reshape(_NH, _E))
    return sco_sorted, tid_sorted, counts


# speedup vs baseline: 1.1166x; 1.1166x over previous
import functools

import jax
import jax.numpy as jnp
from jax import lax
from jax.experimental import pallas as pl
from jax.experimental.pallas import tpu as pltpu
from jax.experimental.pallas import tpu_sc as plsc

_DIM = 2048
_E = 16
_TOKENS = 16384
_FLAT = _TOKENS * 2
_NH = 32
_BT = 2048
_GRID = _TOKENS // _BT
_HPB = _BT // 512

_SC_W = 16
_CHUNK = _FLAT // _SC_W
_COLS = 128
_ROWS = _CHUNK // _COLS


def _router_block(x_ref, wt_ref, bias_ref, pk_ref, hist_ref):
    x = x_ref[...]
    logits = jnp.dot(x, wt_ref[...], preferred_element_type=jnp.float32)
    logits = logits + bias_ref[...]
    eio = lax.broadcasted_iota(jnp.int32, (_BT, _E), 1)
    m1 = jnp.max(logits, axis=1, keepdims=True)
    i1 = jnp.min(jnp.where(logits >= m1, eio, _E), axis=1, keepdims=True)
    masked = jnp.where(eio == i1, -jnp.inf, logits)
    m2 = jnp.max(masked, axis=1, keepdims=True)
    i2 = jnp.min(jnp.where(masked >= m2, eio, _E), axis=1, keepdims=True)
    t = jnp.exp(m2 - m1)
    s = 1.0 / (1.0 + t)
    b1 = lax.bitcast_convert_type(i1, jnp.float32)
    b2 = lax.bitcast_convert_type(i2, jnp.float32)
    pk_ref[...] = jnp.concatenate([s, t * s, b1, b2], axis=1)
    oh = (i1 == eio).astype(jnp.int32) + (i2 == eio).astype(jnp.int32)
    hs = [jnp.sum(oh[c * 512 : (c + 1) * 512], axis=0, keepdims=True)
          for c in range(_HPB)]
    hist_ref[...] = jnp.concatenate(hs, axis=0)[None]


def _router_tc(x, wt, bias2d):
    return pl.pallas_call(
        _router_block,
        grid=(_GRID,),
        in_specs=[
            pl.BlockSpec((_BT, _DIM), lambda i: (i, 0)),
            pl.BlockSpec((_DIM, _E), lambda i: (0, 0)),
            pl.BlockSpec((1, _E), lambda i: (0, 0)),
        ],
        out_specs=[
            pl.BlockSpec((_BT, 4), lambda i: (i, 0)),
            pl.BlockSpec((1, _HPB, _E), lambda i: (i, 0, 0)),
        ],
        out_shape=[
            jax.ShapeDtypeStruct((_TOKENS, 4), jnp.float32),
            jax.ShapeDtypeStruct((_GRID, _HPB, _E), jnp.int32),
        ],
    )(x, wt, bias2d)


def _dispatch_body(pk_hbm, hist_hbm, sco_out, tid_out, cnt_out,
                   pk_v, sco_v, pos_v, tid_v, hist_v, se_b, sl_b, r_b, h_b,
                   rs_b, cnt_v, sco_sh, tid_sh, sem, sem_h, sem_s):
    cid = lax.axis_index("c")
    wid = lax.axis_index("s")
    cp_h = pltpu.make_async_copy(hist_hbm, hist_v, sem_h)
    cp_s = pltpu.make_async_copy(pk_hbm.at[wid], pk_v, sem_s)
    cp_h.start()
    cp_s.start()

    cp_h.wait()
    tot = jnp.zeros((_E,), jnp.int32)
    pre = jnp.zeros((_E,), jnp.int32)
    for c in range(_NH):
        h = hist_v[c // _HPB, c % _HPB, :]
        tot = tot + h
        pre = pre + jnp.where(c < wid * 2, h, 0)
    excl = lax.cumsum(tot, axis=0) - tot

    @pl.when((wid == 0) & (cid == 0))
    def _():
        cnt_v[...] = tot
        pltpu.sync_copy(cnt_v, cnt_out)
    cp_s.wait()

    lane = lax.iota(jnp.int32, 16)
    lane_m1 = jnp.maximum(lane - 1, 0)
    lane_p1 = jnp.minimum(lane + 1, 15)
    flat_base = wid * _CHUNK
    nvr = _CHUNK // 16
    sel_gidx = (lane >> 1) * 4 + 2 + (lane & 1)
    sco_gidx = (lane >> 1) * 4 + (lane & 1)

    @plsc.parallel_loop(0, nvr, unroll=4)
    def _(k):
        off = pl.ds(pl.multiple_of(k * 16, 16), 16)
        e = plsc.bitcast(
            plsc.load_gather(pk_v, [k * 32 + sel_gidx]), jnp.int32)
        sk, sl = plsc.sort_key_val(e * 16 + lane, lane)
        se = sk >> 4
        se_b[off] = se
        k16 = k * 16
        prev = plsc.load_gather(se_b, [k16 + lane_m1])
        nxt = plsc.load_gather(se_b, [k16 + lane_p1])
        is_start = (lane == 0) | (prev != se)
        is_end = (lane == 15) | (nxt != se)
        runstart = plsc.cummax(jnp.where(is_start, lane, 0))
        r = lane - runstart
        sl_b[off] = sl
        r_b[off] = r
        h_b[off] = jnp.zeros((16,), jnp.int32)
        plsc.store_scatter(h_b, [k16 + se], r + 1, mask=is_end)
        tid_v[off] = (flat_base + k16 + lane) >> 1

    def pre_body(k, acc):
        off = pl.ds(pl.multiple_of(k * 16, 16), 16)
        rs_b[off] = acc
        return acc + h_b[off]

    lax.fori_loop(0, nvr, pre_body, excl + pre)

    @plsc.parallel_loop(0, nvr, unroll=4)
    def _(k):
        off = pl.ds(pl.multiple_of(k * 16, 16), 16)
        k16 = k * 16
        se = se_b[off]
        base = plsc.load_gather(rs_b, [k16 + se])
        pos = base + r_b[off]
        row = jnp.zeros((16,), jnp.int32) + (k >> 3)
        col = (k & 7) * 16 + sl_b[off]
        plsc.store_scatter(pos_v, [row, col], pos)

    sl_out = pl.ds(wid * _CHUNK, _CHUNK)

    @pl.when(cid == 0)
    def _():
        @plsc.parallel_loop(0, nvr, unroll=4)
        def _(k):
            off = pl.ds(pl.multiple_of(k * 16, 16), 16)
            sco_v[off] = plsc.load_gather(pk_v, [k * 32 + sco_gidx])

        copies = [pltpu.make_async_copy(
            sco_v.at[pl.ds(j * _COLS, _COLS)], sco_sh.at[pos_v.at[j]], sem)
            for j in range(_ROWS)]
        for cp in copies:
            cp.start()
        for cp in copies:
            cp.wait()
        plsc.subcore_barrier()
        pltpu.sync_copy(sco_sh.at[sl_out], sco_out.at[sl_out])

    @pl.when(cid != 0)
    def _():
        copies = [pltpu.make_async_copy(
            tid_v.at[pl.ds(j * _COLS, _COLS)], tid_sh.at[pos_v.at[j]], sem)
            for j in range(_ROWS)]
        for cp in copies:
            cp.start()
        for cp in copies:
            cp.wait()
        plsc.subcore_barrier()
        pltpu.sync_copy(tid_sh.at[sl_out], tid_out.at[sl_out])


def _dispatch_sc(pk2, hist):
    mesh = plsc.VectorSubcoreMesh(
        core_axis_name="c", subcore_axis_name="s", num_cores=2, num_subcores=16)
    run = pl.kernel(
        _dispatch_body,
        out_type=[
            jax.ShapeDtypeStruct((_FLAT,), jnp.float32),
            jax.ShapeDtypeStruct((_FLAT,), jnp.int32),
            jax.ShapeDtypeStruct((_E,), jnp.int32),
        ],
        mesh=mesh,
        compiler_params=pltpu.CompilerParams(needs_layout_passes=False),
        scratch_types=[
            pltpu.VMEM((2 * _CHUNK,), jnp.float32),
            pltpu.VMEM((_CHUNK,), jnp.float32),
            pltpu.VMEM((_ROWS, _COLS), jnp.int32),
            pltpu.VMEM((_CHUNK,), jnp.int32),
            pltpu.VMEM((_GRID, _HPB, _E), jnp.int32),
            pltpu.VMEM((_CHUNK,), jnp.int32),
            pltpu.VMEM((_CHUNK,), jnp.int32),
            pltpu.VMEM((_CHUNK,), jnp.int32),
            pltpu.VMEM((_CHUNK,), jnp.int32),
            pltpu.VMEM((_CHUNK,), jnp.int32),
            pltpu.VMEM((_E,), jnp.int32),
            pltpu.VMEM_SHARED((_FLAT,), jnp.float32),
            pltpu.VMEM_SHARED((_FLAT,), jnp.int32),
            pltpu.SemaphoreType.DMA,
            pltpu.SemaphoreType.DMA,
            pltpu.SemaphoreType.DMA,
        ],
    )
    return run(pk2, hist)


def kernel(x, expert_bias, W, b):
    wt = W.T
    bias2d = (b + expert_bias).reshape(1, _E)
    pk, hist = _router_tc(x, wt, bias2d)
    pk2 = pk.reshape(_SC_W, 2 * _CHUNK)
    sco_sorted, tid_sorted, counts = _dispatch_sc(pk2, hist)
    return sco_sorted, tid_sorted, counts
